# baseline (speedup 1.0000x reference)
import functools

import jax
import jax.numpy as jnp
from jax.experimental import pallas as pl

_N = 4096
_BLK = 256
_GRID = _N // _BLK
_TBLK = 8
_TGRID = _N // _TBLK

_SORT8 = (
    (0, 1), (2, 3), (4, 5), (6, 7),
    (0, 2), (1, 3), (4, 6), (5, 7),
    (1, 2), (5, 6),
    (0, 4), (1, 5), (2, 6), (3, 7),
    (2, 4), (3, 5),
    (1, 2), (3, 4), (5, 6),
)
_BITONIC8 = (
    (0, 4), (1, 5), (2, 6), (3, 7),
    (0, 2), (1, 3), (4, 6), (5, 7),
    (0, 1), (2, 3), (4, 5), (6, 7),
)


def _cmpex(lists, i, j):
    lo = jnp.minimum(lists[i], lists[j])
    hi = jnp.maximum(lists[i], lists[j])
    lists[i] = lo
    lists[j] = hi


def _top8_desc(prod):
    _, c = prod.shape
    w = c // 8
    lists = [prod[:, t * w:(t + 1) * w] for t in range(8)]
    for i, j in _SORT8:
        _cmpex(lists, i, j)
    while w > 1:
        w //= 2
        a = [l[:, :w] for l in lists]
        b = [l[:, w:] for l in lists]
        lists = [jnp.maximum(a[i], b[7 - i]) for i in range(8)]
        for i, j in _BITONIC8:
            _cmpex(lists, i, j)
    return jnp.concatenate(lists[::-1], axis=1)


def _mm_body(a_ref, b_ref, o_ref):
    o_ref[...] = jnp.dot(a_ref[...], b_ref[...],
                         preferred_element_type=jnp.float32)


def _small_mm(a, b):
    n, _ = a.shape
    m = b.shape[1]
    return pl.pallas_call(
        _mm_body,
        out_shape=jax.ShapeDtypeStruct((n, m), jnp.float32),
    )(a, b)


def _row_mm(adj, b):
    m = b.shape[1]
    return pl.pallas_call(
        _mm_body,
        grid=(_GRID,),
        in_specs=[
            pl.BlockSpec((_BLK, _N), lambda i: (i, 0)),
            pl.BlockSpec((_N, m), lambda i: (0, 0)),
        ],
        out_specs=pl.BlockSpec((_BLK, m), lambda i: (i, 0)),
        out_shape=jax.ShapeDtypeStruct((_N, m), jnp.float32),
    )(adj, b)


def _row_mm_mask_body(adj_ref, b_ref, mask_ref, o_ref):
    o_ref[...] = jnp.dot(adj_ref[...], b_ref[...],
                         preferred_element_type=jnp.float32) * mask_ref[...]


def _row_mm_mask(adj, b, maskf):
    m = b.shape[1]
    return pl.pallas_call(
        _row_mm_mask_body,
        grid=(_GRID,),
        in_specs=[
            pl.BlockSpec((_BLK, _N), lambda i: (i, 0)),
            pl.BlockSpec((_N, m), lambda i: (0, 0)),
            pl.BlockSpec((_BLK, 1), lambda i: (i, 0)),
        ],
        out_specs=pl.BlockSpec((_BLK, m), lambda i: (i, 0)),
        out_shape=jax.ShapeDtypeStruct((_N, m), jnp.float32),
    )(adj, b, maskf)


def _topk_conv_body(adj_ref, h_ref, ht_ref, ck_ref, c0_ref, bias_ref, o_ref,
                    *, nfeat):
    acc = jnp.dot(h_ref[...], c0_ref[...],
                  preferred_element_type=jnp.float32) + bias_ref[...]
    adj = adj_ref[...]

    def body(i, acc):
        fa = 2 * i
        fb = fa + 1
        t8a = _top8_desc(adj * ht_ref[fa])
        t8b = _top8_desc(adj * ht_ref[fb])
        acc = acc + jnp.dot(t8a, ck_ref[fa],
                            preferred_element_type=jnp.float32)
        return acc + jnp.dot(t8b, ck_ref[fb],
                             preferred_element_type=jnp.float32)

    acc = jax.lax.fori_loop(0, nfeat // 2, body, acc)
    o_ref[...] = acc


def _topk_conv(adj, h, cw, bias):
    nfeat = h.shape[1]
    ht = h.T.reshape(nfeat, 1, _N)
    ck = jnp.transpose(cw[1:9], (1, 0, 2))
    c0 = cw[0]
    bias2 = bias.reshape(1, 8)
    return pl.pallas_call(
        functools.partial(_topk_conv_body, nfeat=nfeat),
        grid=(_TGRID,),
        in_specs=[
            pl.BlockSpec((_TBLK, _N), lambda i: (i, 0)),
            pl.BlockSpec((_TBLK, nfeat), lambda i: (i, 0)),
            pl.BlockSpec((nfeat, 1, _N), lambda i: (0, 0, 0)),
            pl.BlockSpec((nfeat, 8, 8), lambda i: (0, 0, 0)),
            pl.BlockSpec((nfeat, 8), lambda i: (0, 0)),
            pl.BlockSpec((1, 8), lambda i: (0, 0)),
        ],
        out_specs=pl.BlockSpec((_TBLK, 8), lambda i: (i, 0)),
        out_shape=jax.ShapeDtypeStruct((_N, 8), jnp.float32),
    )(adj, h, ht, ck, c0, bias2)


def _fuse_conv_weights(wa, wb, gamma):
    kwa, cin, _ = wa.shape
    kwb, _, cout = wb.shape
    cw = jnp.zeros((kwa + kwb - 1, cin, cout), jnp.float32)
    for u in range(kwb):
        for v in range(kwa):
            cw = cw.at[u + v].add(wa[v] @ wb[u])
    scale = gamma / jnp.sqrt(1.0 + 1e-3)
    return cw * scale[None, None, :]


def kernel(x, adj, mask, w1, c1a, c1b, g1, b1, c2a, c2b, g2, b2, w_out):
    cw1 = _fuse_conv_weights(c1a, c1b, g1)
    cw2 = _fuse_conv_weights(c2a, c2b, g2)
    maskf = mask.astype(jnp.float32).reshape(_N, 1)

    h1 = _row_mm(adj, _small_mm(x, w1))
    cur1 = _topk_conv(adj, h1, cw1, b1)
    h2 = jnp.concatenate([h1, cur1], axis=1)
    cur2 = _topk_conv(adj, h2, cw2, b2)
    h3 = jnp.concatenate([h2, cur2], axis=1)
    return _row_mm_mask(adj, _small_mm(h3, w_out), maskf)


# speedup vs baseline: 3.1615x; 3.1615x over previous
import functools

import jax
import jax.numpy as jnp
from jax.experimental import pallas as pl

_N = 4096
_BLK = 256
_GRID = _N // _BLK
_TBLK = 256
_TGRID = _N // _TBLK

_SORT8 = (
    (0, 1), (2, 3), (4, 5), (6, 7),
    (0, 2), (1, 3), (4, 6), (5, 7),
    (1, 2), (5, 6),
    (0, 4), (1, 5), (2, 6), (3, 7),
    (2, 4), (3, 5),
    (1, 2), (3, 4), (5, 6),
)
_BITONIC8 = (
    (0, 4), (1, 5), (2, 6), (3, 7),
    (0, 2), (1, 3), (4, 6), (5, 7),
    (0, 1), (2, 3), (4, 5), (6, 7),
)


def _cmpex(lists, i, j):
    lo = jnp.minimum(lists[i], lists[j])
    hi = jnp.maximum(lists[i], lists[j])
    lists[i] = lo
    lists[j] = hi


def _top8_desc(prod):
    _, c = prod.shape
    w = c // 8
    lists = [prod[:, t * w:(t + 1) * w] for t in range(8)]
    for i, j in _SORT8:
        _cmpex(lists, i, j)
    while w > 1:
        w //= 2
        a = [l[:, :w] for l in lists]
        b = [l[:, w:] for l in lists]
        lists = [jnp.maximum(a[i], b[7 - i]) for i in range(8)]
        for i, j in _BITONIC8:
            _cmpex(lists, i, j)
    return jnp.concatenate(lists[::-1], axis=1)


def _mm_body(a_ref, b_ref, o_ref):
    o_ref[...] = jnp.dot(a_ref[...], b_ref[...],
                         preferred_element_type=jnp.float32)


def _small_mm(a, b):
    n, _ = a.shape
    m = b.shape[1]
    return pl.pallas_call(
        _mm_body,
        out_shape=jax.ShapeDtypeStruct((n, m), jnp.float32),
    )(a, b)


def _row_mm(adj, b):
    m = b.shape[1]
    return pl.pallas_call(
        _mm_body,
        grid=(_GRID,),
        in_specs=[
            pl.BlockSpec((_BLK, _N), lambda i: (i, 0)),
            pl.BlockSpec((_N, m), lambda i: (0, 0)),
        ],
        out_specs=pl.BlockSpec((_BLK, m), lambda i: (i, 0)),
        out_shape=jax.ShapeDtypeStruct((_N, m), jnp.float32),
    )(adj, b)


def _row_mm_mask_body(adj_ref, b_ref, mask_ref, o_ref):
    o_ref[...] = jnp.dot(adj_ref[...], b_ref[...],
                         preferred_element_type=jnp.float32) * mask_ref[...]


def _row_mm_mask(adj, b, maskf):
    m = b.shape[1]
    return pl.pallas_call(
        _row_mm_mask_body,
        grid=(_GRID,),
        in_specs=[
            pl.BlockSpec((_BLK, _N), lambda i: (i, 0)),
            pl.BlockSpec((_N, m), lambda i: (0, 0)),
            pl.BlockSpec((_BLK, 1), lambda i: (i, 0)),
        ],
        out_specs=pl.BlockSpec((_BLK, m), lambda i: (i, 0)),
        out_shape=jax.ShapeDtypeStruct((_N, m), jnp.float32),
    )(adj, b, maskf)


def _topk_conv_body(adj_ref, h_ref, ht_ref, ck_ref, c0_ref, bias_ref, o_ref,
                    *, nfeat):
    acc = jnp.dot(h_ref[...], c0_ref[...],
                  preferred_element_type=jnp.float32) + bias_ref[...]
    adj = adj_ref[...]

    def body(i, acc):
        for k in range(4):
            f = 4 * i + k
            t8 = _top8_desc(adj * ht_ref[f])
            acc = acc + jnp.dot(t8, ck_ref[f],
                                preferred_element_type=jnp.float32)
        return acc

    acc = jax.lax.fori_loop(0, nfeat // 4, body, acc)
    o_ref[...] = acc


def _topk_conv(adj, h, cw, bias):
    nfeat = h.shape[1]
    ht = h.T.reshape(nfeat, 1, _N)
    ck = jnp.transpose(cw[1:9], (1, 0, 2))
    c0 = cw[0]
    bias2 = bias.reshape(1, 8)
    return pl.pallas_call(
        functools.partial(_topk_conv_body, nfeat=nfeat),
        grid=(_TGRID,),
        in_specs=[
            pl.BlockSpec((_TBLK, _N), lambda i: (i, 0)),
            pl.BlockSpec((_TBLK, nfeat), lambda i: (i, 0)),
            pl.BlockSpec((nfeat, 1, _N), lambda i: (0, 0, 0)),
            pl.BlockSpec((nfeat, 8, 8), lambda i: (0, 0, 0)),
            pl.BlockSpec((nfeat, 8), lambda i: (0, 0)),
            pl.BlockSpec((1, 8), lambda i: (0, 0)),
        ],
        out_specs=pl.BlockSpec((_TBLK, 8), lambda i: (i, 0)),
        out_shape=jax.ShapeDtypeStruct((_N, 8), jnp.float32),
    )(adj, h, ht, ck, c0, bias2)


def _fuse_conv_weights(wa, wb, gamma):
    kwa, cin, _ = wa.shape
    kwb, _, cout = wb.shape
    cw = jnp.zeros((kwa + kwb - 1, cin, cout), jnp.float32)
    for u in range(kwb):
        for v in range(kwa):
            cw = cw.at[u + v].add(wa[v] @ wb[u])
    scale = gamma / jnp.sqrt(1.0 + 1e-3)
    return cw * scale[None, None, :]


def kernel(x, adj, mask, w1, c1a, c1b, g1, b1, c2a, c2b, g2, b2, w_out):
    cw1 = _fuse_conv_weights(c1a, c1b, g1)
    cw2 = _fuse_conv_weights(c2a, c2b, g2)
    maskf = mask.astype(jnp.float32).reshape(_N, 1)

    h1 = _row_mm(adj, _small_mm(x, w1))
    cur1 = _topk_conv(adj, h1, cw1, b1)
    h2 = jnp.concatenate([h1, cur1], axis=1)
    cur2 = _topk_conv(adj, h2, cw2, b2)
    h3 = jnp.concatenate([h2, cur2], axis=1)
    return _row_mm_mask(adj, _small_mm(h3, w_out), maskf)


# speedup vs baseline: 6.1045x; 1.9309x over previous
import functools

import jax
import jax.numpy as jnp
from jax.experimental import pallas as pl

_N = 4096
_BLK = 256
_GRID = _N // _BLK
_TL = 256

_SORT8 = (
    (0, 1), (2, 3), (4, 5), (6, 7),
    (0, 2), (1, 3), (4, 6), (5, 7),
    (1, 2), (5, 6),
    (0, 4), (1, 5), (2, 6), (3, 7),
    (2, 4), (3, 5),
    (1, 2), (3, 4), (5, 6),
)
_BITONIC8 = (
    (0, 4), (1, 5), (2, 6), (3, 7),
    (0, 2), (1, 3), (4, 6), (5, 7),
    (0, 1), (2, 3), (4, 5), (6, 7),
)


def _cmpex(lists, i, j):
    lo = jnp.minimum(lists[i], lists[j])
    hi = jnp.maximum(lists[i], lists[j])
    lists[i] = lo
    lists[j] = hi


def _top8_desc_ax0(prod):
    c, _ = prod.shape
    w = c // 8
    lists = [prod[t * w:(t + 1) * w, :] for t in range(8)]
    for i, j in _SORT8:
        _cmpex(lists, i, j)
    while w > 1:
        w //= 2
        a = [l[:w, :] for l in lists]
        b = [l[w:, :] for l in lists]
        lists = [jnp.maximum(a[i], b[7 - i]) for i in range(8)]
        for i, j in _BITONIC8:
            _cmpex(lists, i, j)
    return jnp.concatenate(lists[::-1], axis=0)


def _mm_body(a_ref, b_ref, o_ref):
    o_ref[...] = jnp.dot(a_ref[...], b_ref[...],
                         preferred_element_type=jnp.float32)


def _small_mm(a, b):
    n, _ = a.shape
    m = b.shape[1]
    return pl.pallas_call(
        _mm_body,
        out_shape=jax.ShapeDtypeStruct((n, m), jnp.float32),
    )(a, b)


def _row_mm(adj, b):
    m = b.shape[1]
    return pl.pallas_call(
        _mm_body,
        grid=(_GRID,),
        in_specs=[
            pl.BlockSpec((_BLK, _N), lambda i: (i, 0)),
            pl.BlockSpec((_N, m), lambda i: (0, 0)),
        ],
        out_specs=pl.BlockSpec((_BLK, m), lambda i: (i, 0)),
        out_shape=jax.ShapeDtypeStruct((_N, m), jnp.float32),
    )(adj, b)


def _row_mm_mask_body(adj_ref, b_ref, mask_ref, o_ref):
    o_ref[...] = jnp.dot(adj_ref[...], b_ref[...],
                         preferred_element_type=jnp.float32) * mask_ref[...]


def _row_mm_mask(adj, b, maskf):
    m = b.shape[1]
    return pl.pallas_call(
        _row_mm_mask_body,
        grid=(_GRID,),
        in_specs=[
            pl.BlockSpec((_BLK, _N), lambda i: (i, 0)),
            pl.BlockSpec((_N, m), lambda i: (0, 0)),
            pl.BlockSpec((_BLK, 1), lambda i: (i, 0)),
        ],
        out_specs=pl.BlockSpec((_BLK, m), lambda i: (i, 0)),
        out_shape=jax.ShapeDtypeStruct((_N, m), jnp.float32),
    )(adj, b, maskf)


def _topk_conv_body(adjt_ref, hcol_ref, htb_ref, ckt_ref, c0t_ref, bias_ref,
                    o_ref):
    f = pl.program_id(1)

    @pl.when(f == 0)
    def _init():
        o_ref[...] = jnp.dot(c0t_ref[...], htb_ref[...],
                             preferred_element_type=jnp.float32) + bias_ref[...]

    prod = adjt_ref[...] * hcol_ref[0]
    t8 = _top8_desc_ax0(prod)
    o_ref[...] += jnp.dot(ckt_ref[0], t8, preferred_element_type=jnp.float32)


def _topk_conv(adjt, h, ht, cw, bias):
    nfeat = h.shape[1]
    hc3 = ht.reshape(nfeat, _N, 1)
    ckt = jnp.transpose(cw[1:9], (1, 2, 0))
    c0t = cw[0].T
    bias2 = bias.reshape(8, 1)
    return pl.pallas_call(
        _topk_conv_body,
        grid=(_N // _TL, nfeat),
        in_specs=[
            pl.BlockSpec((_N, _TL), lambda i, f: (0, i)),
            pl.BlockSpec((1, _N, 1), lambda i, f: (f, 0, 0)),
            pl.BlockSpec((nfeat, _TL), lambda i, f: (0, i)),
            pl.BlockSpec((1, 8, 8), lambda i, f: (f, 0, 0)),
            pl.BlockSpec((8, nfeat), lambda i, f: (0, 0)),
            pl.BlockSpec((8, 1), lambda i, f: (0, 0)),
        ],
        out_specs=pl.BlockSpec((8, _TL), lambda i, f: (0, i)),
        out_shape=jax.ShapeDtypeStruct((8, _N), jnp.float32),
    )(adjt, hc3, ht, ckt, c0t, bias2)


def _fuse_conv_weights(wa, wb, gamma):
    kwa, cin, _ = wa.shape
    kwb, _, cout = wb.shape
    cw = jnp.zeros((kwa + kwb - 1, cin, cout), jnp.float32)
    for u in range(kwb):
        for v in range(kwa):
            cw = cw.at[u + v].add(wa[v] @ wb[u])
    scale = gamma / jnp.sqrt(1.0 + 1e-3)
    return cw * scale[None, None, :]


def kernel(x, adj, mask, w1, c1a, c1b, g1, b1, c2a, c2b, g2, b2, w_out):
    cw1 = _fuse_conv_weights(c1a, c1b, g1)
    cw2 = _fuse_conv_weights(c2a, c2b, g2)
    maskf = mask.astype(jnp.float32).reshape(_N, 1)
    adjt = adj.T

    h1 = _row_mm(adj, _small_mm(x, w1))
    h1t = h1.T
    cur1t = _topk_conv(adjt, h1, h1t, cw1, b1)
    h2 = jnp.concatenate([h1, cur1t.T], axis=1)
    h2t = jnp.concatenate([h1t, cur1t], axis=0)
    cur2t = _topk_conv(adjt, h2, h2t, cw2, b2)
    h3 = jnp.concatenate([h2, cur2t.T], axis=1)
    return _row_mm_mask(adj, _small_mm(h3, w_out), maskf)


# speedup vs baseline: 6.6203x; 1.0845x over previous
import functools

import jax
import jax.numpy as jnp
from jax.experimental import pallas as pl

_N = 4096
_BLK = 256
_GRID = _N // _BLK
_TL = 256

_SORT8 = (
    (0, 1), (2, 3), (4, 5), (6, 7),
    (0, 2), (1, 3), (4, 6), (5, 7),
    (1, 2), (5, 6),
    (0, 4), (1, 5), (2, 6), (3, 7),
    (2, 4), (3, 5),
    (1, 2), (3, 4), (5, 6),
)
_BITONIC8 = (
    (0, 4), (1, 5), (2, 6), (3, 7),
    (0, 2), (1, 3), (4, 6), (5, 7),
    (0, 1), (2, 3), (4, 5), (6, 7),
)


def _cmpex(lists, i, j):
    lo = jnp.minimum(lists[i], lists[j])
    hi = jnp.maximum(lists[i], lists[j])
    lists[i] = lo
    lists[j] = hi


def _top8_desc_ax0(prod):
    c, _ = prod.shape
    w = c // 8
    lists = [prod[t * w:(t + 1) * w, :] for t in range(8)]
    for i, j in _SORT8:
        _cmpex(lists, i, j)
    while w > 1:
        w //= 2
        a = [l[:w, :] for l in lists]
        b = [l[w:, :] for l in lists]
        lists = [jnp.maximum(a[i], b[7 - i]) for i in range(8)]
        for i, j in _BITONIC8:
            _cmpex(lists, i, j)
    return jnp.concatenate(lists[::-1], axis=0)


def _mm_body(a_ref, b_ref, o_ref):
    o_ref[...] = jnp.dot(a_ref[...], b_ref[...],
                         preferred_element_type=jnp.float32)


def _small_mm(a, b):
    n, _ = a.shape
    m = b.shape[1]
    return pl.pallas_call(
        _mm_body,
        out_shape=jax.ShapeDtypeStruct((n, m), jnp.float32),
    )(a, b)


def _row_mm(adj, b):
    m = b.shape[1]
    return pl.pallas_call(
        _mm_body,
        grid=(_GRID,),
        in_specs=[
            pl.BlockSpec((_BLK, _N), lambda i: (i, 0)),
            pl.BlockSpec((_N, m), lambda i: (0, 0)),
        ],
        out_specs=pl.BlockSpec((_BLK, m), lambda i: (i, 0)),
        out_shape=jax.ShapeDtypeStruct((_N, m), jnp.float32),
    )(adj, b)


def _row_mm_mask_body(adj_ref, b_ref, mask_ref, o_ref):
    o_ref[...] = jnp.dot(adj_ref[...], b_ref[...],
                         preferred_element_type=jnp.float32) * mask_ref[...]


def _row_mm_mask(adj, b, maskf):
    m = b.shape[1]
    return pl.pallas_call(
        _row_mm_mask_body,
        grid=(_GRID,),
        in_specs=[
            pl.BlockSpec((_BLK, _N), lambda i: (i, 0)),
            pl.BlockSpec((_N, m), lambda i: (0, 0)),
            pl.BlockSpec((_BLK, 1), lambda i: (i, 0)),
        ],
        out_specs=pl.BlockSpec((_BLK, m), lambda i: (i, 0)),
        out_shape=jax.ShapeDtypeStruct((_N, m), jnp.float32),
    )(adj, b, maskf)


def _topk_conv_body(adjt_ref, hcol_ref, htb_ref, ckt_ref, c0t_ref, bias_ref,
                    o_ref):
    f = pl.program_id(1)

    @pl.when(f == 0)
    def _init():
        o_ref[...] = jnp.dot(c0t_ref[...], htb_ref[...],
                             preferred_element_type=jnp.float32) + bias_ref[...]

    adjt = adjt_ref[...]
    acc = o_ref[...]
    for k in range(2):
        prod = adjt * hcol_ref[k]
        t8 = _top8_desc_ax0(prod)
        acc = acc + jnp.dot(ckt_ref[k], t8, preferred_element_type=jnp.float32)
    o_ref[...] = acc


def _topk_conv(adjt, h, ht, cw, bias):
    nfeat = h.shape[1]
    hc3 = ht.reshape(nfeat, _N, 1)
    ckt = jnp.transpose(cw[1:9], (1, 2, 0))
    c0t = cw[0].T
    bias2 = bias.reshape(8, 1)
    return pl.pallas_call(
        _topk_conv_body,
        grid=(_N // _TL, nfeat // 2),
        in_specs=[
            pl.BlockSpec((_N, _TL), lambda i, f: (0, i)),
            pl.BlockSpec((2, _N, 1), lambda i, f: (f, 0, 0)),
            pl.BlockSpec((nfeat, _TL), lambda i, f: (0, i)),
            pl.BlockSpec((2, 8, 8), lambda i, f: (f, 0, 0)),
            pl.BlockSpec((8, nfeat), lambda i, f: (0, 0)),
            pl.BlockSpec((8, 1), lambda i, f: (0, 0)),
        ],
        out_specs=pl.BlockSpec((8, _TL), lambda i, f: (0, i)),
        out_shape=jax.ShapeDtypeStruct((8, _N), jnp.float32),
    )(adjt, hc3, ht, ckt, c0t, bias2)


def _fuse_conv_weights(wa, wb, gamma):
    kwa, cin, _ = wa.shape
    kwb, _, cout = wb.shape
    cw = jnp.zeros((kwa + kwb - 1, cin, cout), jnp.float32)
    for u in range(kwb):
        for v in range(kwa):
            cw = cw.at[u + v].add(wa[v] @ wb[u])
    scale = gamma / jnp.sqrt(1.0 + 1e-3)
    return cw * scale[None, None, :]


def kernel(x, adj, mask, w1, c1a, c1b, g1, b1, c2a, c2b, g2, b2, w_out):
    cw1 = _fuse_conv_weights(c1a, c1b, g1)
    cw2 = _fuse_conv_weights(c2a, c2b, g2)
    maskf = mask.astype(jnp.float32).reshape(_N, 1)
    adjt = adj.T

    h1 = _row_mm(adj, _small_mm(x, w1))
    h1t = h1.T
    cur1t = _topk_conv(adjt, h1, h1t, cw1, b1)
    h2 = jnp.concatenate([h1, cur1t.T], axis=1)
    h2t = jnp.concatenate([h1t, cur1t], axis=0)
    cur2t = _topk_conv(adjt, h2, h2t, cw2, b2)
    h3 = jnp.concatenate([h2, cur2t.T], axis=1)
    return _row_mm_mask(adj, _small_mm(h3, w_out), maskf)


# speedup vs baseline: 7.2781x; 1.0994x over previous
import functools

import jax
import jax.numpy as jnp
from jax.experimental import pallas as pl

_N = 4096
_BLK = 256
_GRID = _N // _BLK
_TL = 512

_SORT8 = (
    (0, 1), (2, 3), (4, 5), (6, 7),
    (0, 2), (1, 3), (4, 6), (5, 7),
    (1, 2), (5, 6),
    (0, 4), (1, 5), (2, 6), (3, 7),
    (2, 4), (3, 5),
    (1, 2), (3, 4), (5, 6),
)
_BITONIC8 = (
    (0, 4), (1, 5), (2, 6), (3, 7),
    (0, 2), (1, 3), (4, 6), (5, 7),
    (0, 1), (2, 3), (4, 5), (6, 7),
)


def _cmpex(lists, i, j):
    lo = jnp.minimum(lists[i], lists[j])
    hi = jnp.maximum(lists[i], lists[j])
    lists[i] = lo
    lists[j] = hi


def _top8_desc_ax0(prod):
    c, _ = prod.shape
    w = c // 8
    lists = [prod[t * w:(t + 1) * w, :] for t in range(8)]
    for i, j in _SORT8:
        _cmpex(lists, i, j)
    while w > 1:
        w //= 2
        a = [l[:w, :] for l in lists]
        b = [l[w:, :] for l in lists]
        lists = [jnp.maximum(a[i], b[7 - i]) for i in range(8)]
        for i, j in _BITONIC8:
            _cmpex(lists, i, j)
    return jnp.concatenate(lists[::-1], axis=0)


def _mm_body(a_ref, b_ref, o_ref):
    o_ref[...] = jnp.dot(a_ref[...], b_ref[...],
                         preferred_element_type=jnp.float32)


def _small_mm(a, b):
    n, _ = a.shape
    m = b.shape[1]
    return pl.pallas_call(
        _mm_body,
        out_shape=jax.ShapeDtypeStruct((n, m), jnp.float32),
    )(a, b)


def _row_mm(adj, b):
    m = b.shape[1]
    return pl.pallas_call(
        _mm_body,
        grid=(_GRID,),
        in_specs=[
            pl.BlockSpec((_BLK, _N), lambda i: (i, 0)),
            pl.BlockSpec((_N, m), lambda i: (0, 0)),
        ],
        out_specs=pl.BlockSpec((_BLK, m), lambda i: (i, 0)),
        out_shape=jax.ShapeDtypeStruct((_N, m), jnp.float32),
    )(adj, b)


def _row_mm_mask_body(adj_ref, b_ref, mask_ref, o_ref):
    o_ref[...] = jnp.dot(adj_ref[...], b_ref[...],
                         preferred_element_type=jnp.float32) * mask_ref[...]


def _row_mm_mask(adj, b, maskf):
    m = b.shape[1]
    return pl.pallas_call(
        _row_mm_mask_body,
        grid=(_GRID,),
        in_specs=[
            pl.BlockSpec((_BLK, _N), lambda i: (i, 0)),
            pl.BlockSpec((_N, m), lambda i: (0, 0)),
            pl.BlockSpec((_BLK, 1), lambda i: (i, 0)),
        ],
        out_specs=pl.BlockSpec((_BLK, m), lambda i: (i, 0)),
        out_shape=jax.ShapeDtypeStruct((_N, m), jnp.float32),
    )(adj, b, maskf)


def _topk_conv_body(adjt_ref, hcol_ref, htb_ref, ckt_ref, c0t_ref, bias_ref,
                    o_ref):
    f = pl.program_id(1)

    @pl.when(f == 0)
    def _init():
        o_ref[...] = jnp.dot(c0t_ref[...], htb_ref[...],
                             preferred_element_type=jnp.float32) + bias_ref[...]

    adjt = adjt_ref[...]
    acc = o_ref[...]
    for k in range(2):
        prod = adjt * hcol_ref[k]
        t8 = _top8_desc_ax0(prod)
        acc = acc + jnp.dot(ckt_ref[k], t8, preferred_element_type=jnp.float32)
    o_ref[...] = acc


def _topk_conv(adjt, h, ht, cw, bias):
    nfeat = h.shape[1]
    hc3 = ht.reshape(nfeat, _N, 1)
    ckt = jnp.transpose(cw[1:9], (1, 2, 0))
    c0t = cw[0].T
    bias2 = bias.reshape(8, 1)
    return pl.pallas_call(
        _topk_conv_body,
        grid=(_N // _TL, nfeat // 2),
        in_specs=[
            pl.BlockSpec((_N, _TL), lambda i, f: (0, i)),
            pl.BlockSpec((2, _N, 1), lambda i, f: (f, 0, 0)),
            pl.BlockSpec((nfeat, _TL), lambda i, f: (0, i)),
            pl.BlockSpec((2, 8, 8), lambda i, f: (f, 0, 0)),
            pl.BlockSpec((8, nfeat), lambda i, f: (0, 0)),
            pl.BlockSpec((8, 1), lambda i, f: (0, 0)),
        ],
        out_specs=pl.BlockSpec((8, _TL), lambda i, f: (0, i)),
        out_shape=jax.ShapeDtypeStruct((8, _N), jnp.float32),
    )(adjt, hc3, ht, ckt, c0t, bias2)


def _fuse_conv_weights(wa, wb, gamma):
    kwa, cin, _ = wa.shape
    kwb, _, cout = wb.shape
    cw = jnp.zeros((kwa + kwb - 1, cin, cout), jnp.float32)
    for u in range(kwb):
        for v in range(kwa):
            cw = cw.at[u + v].add(wa[v] @ wb[u])
    scale = gamma / jnp.sqrt(1.0 + 1e-3)
    return cw * scale[None, None, :]


def kernel(x, adj, mask, w1, c1a, c1b, g1, b1, c2a, c2b, g2, b2, w_out):
    cw1 = _fuse_conv_weights(c1a, c1b, g1)
    cw2 = _fuse_conv_weights(c2a, c2b, g2)
    maskf = mask.astype(jnp.float32).reshape(_N, 1)
    adjt = adj.T

    h1 = _row_mm(adj, _small_mm(x, w1))
    h1t = h1.T
    cur1t = _topk_conv(adjt, h1, h1t, cw1, b1)
    h2 = jnp.concatenate([h1, cur1t.T], axis=1)
    h2t = jnp.concatenate([h1t, cur1t], axis=0)
    cur2t = _topk_conv(adjt, h2, h2t, cw2, b2)
    h3 = jnp.concatenate([h2, cur2t.T], axis=1)
    return _row_mm_mask(adj, _small_mm(h3, w_out), maskf)


# speedup vs baseline: 7.3596x; 1.0112x over previous
import functools

import jax
import jax.numpy as jnp
from jax.experimental import pallas as pl

_N = 4096
_BLK = 256
_GRID = _N // _BLK
_TL = 512

_SORT8 = (
    (0, 1), (2, 3), (4, 5), (6, 7),
    (0, 2), (1, 3), (4, 6), (5, 7),
    (1, 2), (5, 6),
    (0, 4), (1, 5), (2, 6), (3, 7),
    (2, 4), (3, 5),
    (1, 2), (3, 4), (5, 6),
)
_BITONIC8 = (
    (0, 4), (1, 5), (2, 6), (3, 7),
    (0, 2), (1, 3), (4, 6), (5, 7),
    (0, 1), (2, 3), (4, 5), (6, 7),
)


def _cmpex(lists, i, j):
    lo = jnp.minimum(lists[i], lists[j])
    hi = jnp.maximum(lists[i], lists[j])
    lists[i] = lo
    lists[j] = hi


def _top8_desc_ax0(prod):
    c, _ = prod.shape
    w = c // 8
    lists = [prod[t * w:(t + 1) * w, :] for t in range(8)]
    for i, j in _SORT8:
        _cmpex(lists, i, j)
    while w > 1:
        w //= 2
        a = [l[:w, :] for l in lists]
        b = [l[w:, :] for l in lists]
        lists = [jnp.maximum(a[i], b[7 - i]) for i in range(8)]
        for i, j in _BITONIC8:
            _cmpex(lists, i, j)
    return jnp.concatenate(lists[::-1], axis=0)


def _mm_body(a_ref, b_ref, o_ref):
    o_ref[...] = jnp.dot(a_ref[...], b_ref[...],
                         preferred_element_type=jnp.float32)


def _small_mm(a, b):
    n, _ = a.shape
    m = b.shape[1]
    return pl.pallas_call(
        _mm_body,
        out_shape=jax.ShapeDtypeStruct((n, m), jnp.float32),
    )(a, b)


def _row_mm(adj, b):
    m = b.shape[1]
    return pl.pallas_call(
        _mm_body,
        grid=(_GRID,),
        in_specs=[
            pl.BlockSpec((_BLK, _N), lambda i: (i, 0)),
            pl.BlockSpec((_N, m), lambda i: (0, 0)),
        ],
        out_specs=pl.BlockSpec((_BLK, m), lambda i: (i, 0)),
        out_shape=jax.ShapeDtypeStruct((_N, m), jnp.float32),
    )(adj, b)


def _row_mm_mask_body(adj_ref, b_ref, mask_ref, o_ref):
    o_ref[...] = jnp.dot(adj_ref[...], b_ref[...],
                         preferred_element_type=jnp.float32) * mask_ref[...]


def _row_mm_mask(adj, b, maskf):
    m = b.shape[1]
    return pl.pallas_call(
        _row_mm_mask_body,
        grid=(_GRID,),
        in_specs=[
            pl.BlockSpec((_BLK, _N), lambda i: (i, 0)),
            pl.BlockSpec((_N, m), lambda i: (0, 0)),
            pl.BlockSpec((_BLK, 1), lambda i: (i, 0)),
        ],
        out_specs=pl.BlockSpec((_BLK, m), lambda i: (i, 0)),
        out_shape=jax.ShapeDtypeStruct((_N, m), jnp.float32),
    )(adj, b, maskf)


def _topk_conv_body(adjt_ref, hcol_ref, htb_ref, ckt_ref, c0t_ref, bias_ref,
                    o_ref):
    f = pl.program_id(1)

    @pl.when(f == 0)
    def _init():
        o_ref[...] = jnp.dot(c0t_ref[...], htb_ref[...],
                             preferred_element_type=jnp.float32) + bias_ref[...]

    adjt = adjt_ref[...]
    acc = o_ref[...]
    for k in range(4):
        prod = adjt * hcol_ref[k]
        t8 = _top8_desc_ax0(prod)
        acc = acc + jnp.dot(ckt_ref[k], t8, preferred_element_type=jnp.float32)
    o_ref[...] = acc


def _topk_conv(adjt, h, ht, cw, bias):
    nfeat = h.shape[1]
    hc3 = ht.reshape(nfeat, _N, 1)
    ckt = jnp.transpose(cw[1:9], (1, 2, 0))
    c0t = cw[0].T
    bias2 = bias.reshape(8, 1)
    return pl.pallas_call(
        _topk_conv_body,
        grid=(_N // _TL, nfeat // 4),
        in_specs=[
            pl.BlockSpec((_N, _TL), lambda i, f: (0, i)),
            pl.BlockSpec((4, _N, 1), lambda i, f: (f, 0, 0)),
            pl.BlockSpec((nfeat, _TL), lambda i, f: (0, i)),
            pl.BlockSpec((4, 8, 8), lambda i, f: (f, 0, 0)),
            pl.BlockSpec((8, nfeat), lambda i, f: (0, 0)),
            pl.BlockSpec((8, 1), lambda i, f: (0, 0)),
        ],
        out_specs=pl.BlockSpec((8, _TL), lambda i, f: (0, i)),
        out_shape=jax.ShapeDtypeStruct((8, _N), jnp.float32),
    )(adjt, hc3, ht, ckt, c0t, bias2)


def _fuse_conv_weights(wa, wb, gamma):
    kwa, cin, _ = wa.shape
    kwb, _, cout = wb.shape
    cw = jnp.zeros((kwa + kwb - 1, cin, cout), jnp.float32)
    for u in range(kwb):
        for v in range(kwa):
            cw = cw.at[u + v].add(wa[v] @ wb[u])
    scale = gamma / jnp.sqrt(1.0 + 1e-3)
    return cw * scale[None, None, :]


def kernel(x, adj, mask, w1, c1a, c1b, g1, b1, c2a, c2b, g2, b2, w_out):
    cw1 = _fuse_conv_weights(c1a, c1b, g1)
    cw2 = _fuse_conv_weights(c2a, c2b, g2)
    maskf = mask.astype(jnp.float32).reshape(_N, 1)
    adjt = adj.T

    h1 = _row_mm(adj, _small_mm(x, w1))
    h1t = h1.T
    cur1t = _topk_conv(adjt, h1, h1t, cw1, b1)
    h2 = jnp.concatenate([h1, cur1t.T], axis=1)
    h2t = jnp.concatenate([h1t, cur1t], axis=0)
    cur2t = _topk_conv(adjt, h2, h2t, cw2, b2)
    h3 = jnp.concatenate([h2, cur2t.T], axis=1)
    return _row_mm_mask(adj, _small_mm(h3, w_out), maskf)


# speedup vs baseline: 11.9061x; 1.6178x over previous
import functools

import jax
import jax.numpy as jnp
from jax.experimental import pallas as pl

_N = 4096
_BLK = 256
_GRID = _N // _BLK
_TL = 512

_SORT8 = (
    (0, 1), (2, 3), (4, 5), (6, 7),
    (0, 2), (1, 3), (4, 6), (5, 7),
    (1, 2), (5, 6),
    (0, 4), (1, 5), (2, 6), (3, 7),
    (2, 4), (3, 5),
    (1, 2), (3, 4), (5, 6),
)
_BITONIC8 = (
    (0, 4), (1, 5), (2, 6), (3, 7),
    (0, 2), (1, 3), (4, 6), (5, 7),
    (0, 1), (2, 3), (4, 5), (6, 7),
)


def _cmpex(lists, i, j):
    lo = jnp.minimum(lists[i], lists[j])
    hi = jnp.maximum(lists[i], lists[j])
    lists[i] = lo
    lists[j] = hi


def _top8_desc_ax0(prod):
    c, _ = prod.shape
    w = c // 8
    lists = [prod[t * w:(t + 1) * w, :] for t in range(8)]
    for i, j in _SORT8:
        _cmpex(lists, i, j)
    while w > 1:
        w //= 2
        a = [l[:w, :] for l in lists]
        b = [l[w:, :] for l in lists]
        lists = [jnp.maximum(a[i], b[7 - i]) for i in range(8)]
        for i, j in _BITONIC8:
            _cmpex(lists, i, j)
    return jnp.concatenate(lists[::-1], axis=0)


def _mm_body(a_ref, b_ref, o_ref):
    o_ref[...] = jnp.dot(a_ref[...], b_ref[...],
                         preferred_element_type=jnp.float32)


def _small_mm(a, b):
    n, _ = a.shape
    m = b.shape[1]
    return pl.pallas_call(
        _mm_body,
        out_shape=jax.ShapeDtypeStruct((n, m), jnp.float32),
    )(a, b)


def _row_mm(adj, b):
    m = b.shape[1]
    return pl.pallas_call(
        _mm_body,
        grid=(_GRID,),
        in_specs=[
            pl.BlockSpec((_BLK, _N), lambda i: (i, 0)),
            pl.BlockSpec((_N, m), lambda i: (0, 0)),
        ],
        out_specs=pl.BlockSpec((_BLK, m), lambda i: (i, 0)),
        out_shape=jax.ShapeDtypeStruct((_N, m), jnp.float32),
    )(adj, b)


def _row_mm_mask_body(adj_ref, b_ref, mask_ref, o_ref):
    o_ref[...] = jnp.dot(adj_ref[...], b_ref[...],
                         preferred_element_type=jnp.float32) * mask_ref[...]


def _row_mm_mask(adj, b, maskf):
    m = b.shape[1]
    return pl.pallas_call(
        _row_mm_mask_body,
        grid=(_GRID,),
        in_specs=[
            pl.BlockSpec((_BLK, _N), lambda i: (i, 0)),
            pl.BlockSpec((_N, m), lambda i: (0, 0)),
            pl.BlockSpec((_BLK, 1), lambda i: (i, 0)),
        ],
        out_specs=pl.BlockSpec((_BLK, m), lambda i: (i, 0)),
        out_shape=jax.ShapeDtypeStruct((_N, m), jnp.float32),
    )(adj, b, maskf)


def _topk_dual_body(adjt_ref, hcol_ref, htb_ref, ckt1_ref, ckt2_ref,
                    c0t1_ref, c0t2_ref, b1_ref, b2_ref, o1_ref, o2_ref):
    f = pl.program_id(1)

    @pl.when(f == 0)
    def _init():
        htb = htb_ref[...]
        o1_ref[...] = jnp.dot(c0t1_ref[...], htb,
                              preferred_element_type=jnp.float32) + b1_ref[...]
        o2_ref[...] = jnp.dot(c0t2_ref[...], htb,
                              preferred_element_type=jnp.float32) + b2_ref[...]

    adjt = adjt_ref[...]
    acc1 = o1_ref[...]
    acc2 = o2_ref[...]
    for k in range(4):
        prod = adjt * hcol_ref[k]
        t8 = _top8_desc_ax0(prod)
        acc1 = acc1 + jnp.dot(ckt1_ref[k], t8,
                              preferred_element_type=jnp.float32)
        acc2 = acc2 + jnp.dot(ckt2_ref[k], t8,
                              preferred_element_type=jnp.float32)
    o1_ref[...] = acc1
    o2_ref[...] = acc2


def _topk_tail_body(adjt_ref, hcol_ref, curtb_ref, part_ref, ckt_ref,
                    c0t_ref, o_ref):
    f = pl.program_id(1)

    @pl.when(f == 0)
    def _init():
        o_ref[...] = part_ref[...] + jnp.dot(
            c0t_ref[...], curtb_ref[...], preferred_element_type=jnp.float32)

    adjt = adjt_ref[...]
    acc = o_ref[...]
    for k in range(4):
        prod = adjt * hcol_ref[k]
        t8 = _top8_desc_ax0(prod)
        acc = acc + jnp.dot(ckt_ref[k], t8, preferred_element_type=jnp.float32)
    o_ref[...] = acc


def _topk_dual(adjt, ht, cw1, cw2, bias1, bias2):
    nfeat = ht.shape[0]
    hc3 = ht.reshape(nfeat, _N, 1)
    ckt1 = jnp.transpose(cw1[1:9], (1, 2, 0))
    ckt2 = jnp.transpose(cw2[1:9, :nfeat], (1, 2, 0))
    c0t1 = cw1[0].T
    c0t2 = cw2[0, :nfeat].T
    b1 = bias1.reshape(8, 1)
    b2 = bias2.reshape(8, 1)
    return pl.pallas_call(
        _topk_dual_body,
        grid=(_N // _TL, nfeat // 4),
        in_specs=[
            pl.BlockSpec((_N, _TL), lambda i, f: (0, i)),
            pl.BlockSpec((4, _N, 1), lambda i, f: (f, 0, 0)),
            pl.BlockSpec((nfeat, _TL), lambda i, f: (0, i)),
            pl.BlockSpec((4, 8, 8), lambda i, f: (f, 0, 0)),
            pl.BlockSpec((4, 8, 8), lambda i, f: (f, 0, 0)),
            pl.BlockSpec((8, nfeat), lambda i, f: (0, 0)),
            pl.BlockSpec((8, nfeat), lambda i, f: (0, 0)),
            pl.BlockSpec((8, 1), lambda i, f: (0, 0)),
            pl.BlockSpec((8, 1), lambda i, f: (0, 0)),
        ],
        out_specs=[
            pl.BlockSpec((8, _TL), lambda i, f: (0, i)),
            pl.BlockSpec((8, _TL), lambda i, f: (0, i)),
        ],
        out_shape=[
            jax.ShapeDtypeStruct((8, _N), jnp.float32),
            jax.ShapeDtypeStruct((8, _N), jnp.float32),
        ],
    )(adjt, hc3, ht, ckt1, ckt2, c0t1, c0t2, b1, b2)


def _topk_tail(adjt, curt, part2, cw2):
    nf2 = cw2.shape[1]
    hc3 = curt.reshape(8, _N, 1)
    ckt = jnp.transpose(cw2[1:9, nf2 - 8:], (1, 2, 0))
    c0t = cw2[0, nf2 - 8:].T
    return pl.pallas_call(
        _topk_tail_body,
        grid=(_N // _TL, 2),
        in_specs=[
            pl.BlockSpec((_N, _TL), lambda i, f: (0, i)),
            pl.BlockSpec((4, _N, 1), lambda i, f: (f, 0, 0)),
            pl.BlockSpec((8, _TL), lambda i, f: (0, i)),
            pl.BlockSpec((8, _TL), lambda i, f: (0, i)),
            pl.BlockSpec((4, 8, 8), lambda i, f: (f, 0, 0)),
            pl.BlockSpec((8, 8), lambda i, f: (0, 0)),
        ],
        out_specs=pl.BlockSpec((8, _TL), lambda i, f: (0, i)),
        out_shape=jax.ShapeDtypeStruct((8, _N), jnp.float32),
    )(adjt, hc3, curt, part2, ckt, c0t)


def _fuse_conv_weights(wa, wb, gamma):
    kwa, cin, _ = wa.shape
    kwb, _, cout = wb.shape
    cw = jnp.zeros((kwa + kwb - 1, cin, cout), jnp.float32)
    for u in range(kwb):
        for v in range(kwa):
            cw = cw.at[u + v].add(wa[v] @ wb[u])
    scale = gamma / jnp.sqrt(1.0 + 1e-3)
    return cw * scale[None, None, :]


def kernel(x, adj, mask, w1, c1a, c1b, g1, b1, c2a, c2b, g2, b2, w_out):
    cw1 = _fuse_conv_weights(c1a, c1b, g1)
    cw2 = _fuse_conv_weights(c2a, c2b, g2)
    maskf = mask.astype(jnp.float32).reshape(_N, 1)
    adjt = adj.T

    h1 = _row_mm(adj, _small_mm(x, w1))
    h1t = h1.T
    cur1t, part2 = _topk_dual(adjt, h1t, cw1, cw2, b1, b2)
    cur2t = _topk_tail(adjt, cur1t, part2, cw2)
    h3 = jnp.concatenate([h1, cur1t.T, cur2t.T], axis=1)
    return _row_mm_mask(adj, _small_mm(h3, w_out), maskf)
